# baseline (speedup 1.0000x reference)
import functools

import jax
import jax.numpy as jnp
from jax import lax
from jax.experimental import pallas as pl
from jax.experimental.pallas import tpu as pltpu
from jax.experimental.pallas import tpu_sc as plsc

NB = 2000
NI = 4000
NC = 500
E = 64

BLK_I = 800
BLK_P = 800
BLK_PB = 400

NI_PAD = 4096

F32 = jnp.float32
BF16 = jnp.bfloat16
F8 = jnp.float8_e4m3fn


def _split_hi_lo(x):
    hi = x.astype(BF16)
    lo = (x - hi.astype(F32)).astype(BF16)
    return hi, lo


def _bdot(a_bf16, x_f32):
    hi, lo = _split_hi_lo(x_f32)
    r = jnp.dot(a_bf16, hi, preferred_element_type=F32)
    r += jnp.dot(a_bf16, lo, preferred_element_type=F32)
    return r


def _l2n(x):
    n = jnp.sqrt(jnp.sum(x * x, axis=1, keepdims=True))
    return x / jnp.maximum(n, 1e-12)


def _inv_sqrt_deg(d):
    return 1.0 / (jnp.sqrt(d) + 1e-8)


def _k1a_body(bi8_ref, biT8_ref, ii_ref, di_ref, biTbf_ref):
    i = pl.program_id(0)
    biT_blk = biT8_ref[pl.ds(i * BLK_I, BLK_I), :]
    biTbf = biT_blk.astype(BF16)
    biTbf_ref[...] = biTbf
    cnt_i = jnp.dot(biT_blk, bi8_ref[...], preferred_element_type=F32)
    bin_i = jnp.minimum(cnt_i, 1.0)
    ii_ref[...] = bin_i.astype(BF16)
    deg_i = jnp.sum(bin_i, axis=1, keepdims=True)
    deg_i += jnp.dot(biTbf, jnp.ones((NB, 1), BF16),
                     preferred_element_type=F32)
    di_ref[pl.ds(i * BLK_I, BLK_I), :] = deg_i


def _k1a(bi8, biT8):
    full = lambda shape: pl.BlockSpec(shape, lambda i: tuple(0 for _ in shape))
    return pl.pallas_call(
        _k1a_body,
        grid=(NI // BLK_I,),
        in_specs=[full((NB, NI)), full((NI, NB))],
        out_specs=[
            pl.BlockSpec((BLK_I, NI), lambda i: (i, 0)),
            full((NI, 1)),
            pl.BlockSpec((BLK_I, NB), lambda i: (i, 0)),
        ],
        out_shape=[
            jax.ShapeDtypeStruct((NI, NI), BF16),
            jax.ShapeDtypeStruct((NI, 1), F32),
            jax.ShapeDtypeStruct((NI, NB), BF16),
        ],
    )(bi8, biT8)


def _k1b_body(bi8_ref, bb_ref, biT8_ref, db_ref):
    biT8 = jnp.swapaxes(bi8_ref[...], 0, 1)
    biT8_ref[...] = biT8
    cnt_b = jnp.dot(bi8_ref[...], biT8, preferred_element_type=F32)
    bin_b = jnp.minimum(cnt_b, 1.0)
    bb_ref[...] = bin_b.astype(BF16)
    deg_b = jnp.sum(bin_b, axis=1, keepdims=True)
    deg_b += jnp.dot(bi8_ref[...], jnp.ones((NI, 1), F8),
                     preferred_element_type=F32)
    db_ref[...] = deg_b


def _k1b(bi8):
    return pl.pallas_call(
        _k1b_body,
        out_shape=[
            jax.ShapeDtypeStruct((NB, NB), BF16),
            jax.ShapeDtypeStruct((NI, NB), F8),
            jax.ShapeDtypeStruct((NB, 1), F32),
        ],
    )(bi8)


def _k1c_body(bi8_ref, ic8_ref, bb_ref, db_ref, di_ref, fb_ref, fc_ref,
              fi_ref, clc_ref, cid_ref, pb_ref, u0b_ref, u0i_ref):
    bi8 = bi8_ref[...]
    ic8 = ic8_ref[...]
    bc = jnp.dot(bi8, ic8, preferred_element_type=F32)
    bcT = lax.dot_general(ic8, bi8, (((0,), (1,)), ((), ())),
                          preferred_element_type=F32)
    dbc = jnp.sum(bc, axis=1, keepdims=True)
    dc = jnp.sum(bcT, axis=1, keepdims=True)
    sbc = _inv_sqrt_deg(dbc)
    sc = _inv_sqrt_deg(dc)
    fb = fb_ref[...]
    fc = fc_ref[...]
    f1b = sbc * jnp.dot(bc, sc * fc, preferred_element_type=F32) * 0.5
    f1c = sc * jnp.dot(bcT, sbc * fb, preferred_element_type=F32) * 0.5
    f2c = sc * jnp.dot(bcT, sbc * f1b, preferred_element_type=F32) * (1.0 / 3.0)
    clc = fc + _l2n(f1c) + _l2n(f2c)
    clc_ref[...] = jnp.concatenate([clc, jnp.zeros((NC, 128 - E), F32)],
                                   axis=1)
    iota = lax.broadcasted_iota(jnp.int32, (NC, 1), 0).astype(F32)
    cid = _bdot(ic8.astype(BF16), iota)
    cid_ref[...] = cid.astype(jnp.int32)
    sb = _inv_sqrt_deg(db_ref[...])
    si = _inv_sqrt_deg(di_ref[...])
    u0b = sb * fb
    u0i = si * fi_ref[...]
    u0b_ref[...] = u0b
    u0i_ref[...] = u0i
    pb_ref[...] = _bdot(bb_ref[...], u0b)


def _k1c(bi8, ic8, bb, db, di, fb, fc, fi):
    return pl.pallas_call(
        _k1c_body,
        out_shape=[
            jax.ShapeDtypeStruct((NC, 128), F32),
            jax.ShapeDtypeStruct((NI, 1), jnp.int32),
            jax.ShapeDtypeStruct((NB, E), F32),
            jax.ShapeDtypeStruct((NB, E), F32),
            jax.ShapeDtypeStruct((NI, E), F32),
        ],
    )(bi8, ic8, bb, db, di, fb, fc, fi)


def _sc_gather_body(clc_hbm, cid_hbm, cli_hbm, idx_v, rows_v, sem):
    info = plsc.get_sparse_core_info()
    wid = lax.axis_index("s") * info.num_cores + lax.axis_index("c")
    base = wid * (NI_PAD // 32)
    pltpu.sync_copy(cid_hbm.at[pl.ds(base, NI_PAD // 32)], idx_v)
    pltpu.async_copy(clc_hbm.at[idx_v], rows_v, sem).wait()
    pltpu.sync_copy(rows_v, cli_hbm.at[pl.ds(base, NI_PAD // 32)])


def _sc_gather(clc, cid_flat):
    mesh = plsc.VectorSubcoreMesh(core_axis_name="c", subcore_axis_name="s")
    k = functools.partial(
        pl.kernel,
        mesh=mesh,
        out_type=jax.ShapeDtypeStruct((NI_PAD, 128), F32),
        scratch_types=[
            pltpu.VMEM((NI_PAD // 32,), jnp.int32),
            pltpu.VMEM((NI_PAD // 32, 128), F32),
            pltpu.SemaphoreType.DMA,
        ],
    )(_sc_gather_body)
    return k(clc, cid_flat)


def _bdot_t(a_bf16, x_f32):
    hi, lo = _split_hi_lo(x_f32)
    dn = (((0,), (0,)), ((), ()))
    r = lax.dot_general(a_bf16, hi, dn, preferred_element_type=F32)
    r += lax.dot_general(a_bf16, lo, dn, preferred_element_type=F32)
    return r


def _k2a_body(biT_ref, ii_ref, dib_ref, u0b_ref, u0i_ref, u0ib_ref,
              db_ref, pb_ref, n1i_ref, u1i_ref, u1b_ref, acc_scr):
    j = pl.program_id(0)
    sib = _inv_sqrt_deg(dib_ref[...])
    biT_blk = biT_ref[...]
    f1i = sib * (_bdot(biT_blk, u0b_ref[...])
                 + _bdot(ii_ref[...], u0i_ref[...])) * 0.5
    n1i_ref[...] = _l2n(f1i)
    u1i_ref[...] = sib * f1i

    part = _bdot_t(biT_blk, u0ib_ref[...])

    @pl.when(j == 0)
    def _():
        acc_scr[...] = part

    @pl.when(j > 0)
    def _():
        acc_scr[...] += part

    @pl.when(j == NI // BLK_P - 1)
    def _():
        sb = _inv_sqrt_deg(db_ref[...])
        u1b_ref[...] = 0.5 * sb * sb * (pb_ref[...] + acc_scr[...])


def _k2a(biT_bf, ii, di, u0b, u0i, db, pb):
    return pl.pallas_call(
        _k2a_body,
        grid=(NI // BLK_P,),
        in_specs=[
            pl.BlockSpec((BLK_P, NB), lambda j: (j, 0)),
            pl.BlockSpec((BLK_P, NI), lambda j: (j, 0)),
            pl.BlockSpec((BLK_P, 1), lambda j: (j, 0)),
            pl.BlockSpec((NB, E), lambda j: (0, 0)),
            pl.BlockSpec((NI, E), lambda j: (0, 0)),
            pl.BlockSpec((BLK_P, E), lambda j: (j, 0)),
            pl.BlockSpec((NB, 1), lambda j: (0, 0)),
            pl.BlockSpec((NB, E), lambda j: (0, 0)),
        ],
        out_specs=[
            pl.BlockSpec((BLK_P, E), lambda j: (j, 0)),
            pl.BlockSpec((BLK_P, E), lambda j: (j, 0)),
            pl.BlockSpec((NB, E), lambda j: (0, 0)),
        ],
        out_shape=[
            jax.ShapeDtypeStruct((NI, E), F32),
            jax.ShapeDtypeStruct((NI, E), F32),
            jax.ShapeDtypeStruct((NB, E), F32),
        ],
        scratch_shapes=[pltpu.VMEM((NB, E), F32)],
    )(biT_bf, ii, di, u0b, u0i, u0i, db, pb)


def _k2b_body(biT_ref, ii_ref, u1b_ref, u1i_ref, dib_ref, fi_ref, n1i_ref,
              cli_ref, out_ref):
    sib = _inv_sqrt_deg(dib_ref[...])
    f2i = sib * (_bdot(biT_ref[...], u1b_ref[...])
                 + _bdot(ii_ref[...], u1i_ref[...])) * (1.0 / 3.0)
    il = fi_ref[...] + n1i_ref[...] + _l2n(f2i)
    out_ref[...] = cli_ref[...] * 0.6 + il * 0.4


def _k2b(biT_bf, ii, u1b, u1i, di, fi, n1i, cli):
    return pl.pallas_call(
        _k2b_body,
        grid=(NI // BLK_P,),
        in_specs=[
            pl.BlockSpec((BLK_P, NB), lambda j: (j, 0)),
            pl.BlockSpec((BLK_P, NI), lambda j: (j, 0)),
            pl.BlockSpec((NB, E), lambda j: (0, 0)),
            pl.BlockSpec((NI, E), lambda j: (0, 0)),
            pl.BlockSpec((BLK_P, 1), lambda j: (j, 0)),
            pl.BlockSpec((BLK_P, E), lambda j: (j, 0)),
            pl.BlockSpec((BLK_P, E), lambda j: (j, 0)),
            pl.BlockSpec((BLK_P, E), lambda j: (j, 0)),
        ],
        out_specs=pl.BlockSpec((BLK_P, E), lambda j: (j, 0)),
        out_shape=jax.ShapeDtypeStruct((NI, E), F32),
    )(biT_bf, ii, u1b, u1i, di, fi, n1i, cli)


def kernel(bi_graph, ic_graph, bundles_feature, cates_feature, items_feature):
    bi8 = bi_graph.astype(F8)
    ic8 = ic_graph.astype(F8)

    bb, biT8, db = _k1b(bi8)
    ii, di, biT_bf = _k1a(bi8, biT8)
    clc, cid, pb, u0b, u0i = _k1c(bi8, ic8, bb, db, di, bundles_feature,
                                  cates_feature, items_feature)

    cid_flat = jnp.pad(cid[:, 0], (0, NI_PAD - NI))
    cli = _sc_gather(clc, cid_flat)[:NI, :E]

    n1i, u1i, u1b = _k2a(biT_bf, ii, di, u0b, u0i, db, pb)
    out = _k2b(biT_bf, ii, u1b, u1i, di, items_feature, n1i, cli)
    return out
